# baseline (speedup 1.0000x reference)
import jax
import jax.numpy as jnp
from jax import lax
from jax.experimental import pallas as pl
from jax.experimental.pallas import tpu as pltpu
from jax.experimental.pallas import tpu_sc as plsc

H = 128
NN = 10000
NE = 320000
NC, NS, L = 2, 16, 16
NW = NC * NS

EPW = NE // NW
EC = 128
ECF = EPW // EC
ECT = EPW - ECF * EC
EB = 2000
NB = 2

NT = 25
NPW = NN // NT
NCC = 80
NCH = NPW // NCC

ETROWS = 22 * 6 * 2
NTROWS = 476 + 99 + 108 + 40
OFF1, OFF2, OFF3 = 476, 575, 683


def _iota16():
    return lax.iota(jnp.int32, L)


def _sc_body(x_t, ea_t, ntab, etab, node_out, edge_out,
             etab_v, ntab_s, eidx, ea_c, erows0, erows1, erows2, x_v, nidx,
             nb0, nb1, nb2, nb3,
             ws0, ws1, ws2, gs0, gs1, gs2, ns0, ns1, ns2, ns3):
    erows = (erows0, erows1, erows2)
    wsems = (ws0, ws1, ws2)
    gsems = (gs0, gs1, gs2)
    nbuf = (nb0, nb1, nb2, nb3)
    nsem = (ns0, ns1, ns2, ns3)
    wid = lax.axis_index("s") * NC + lax.axis_index("c")

    @pl.when(lax.axis_index("s") == 0)
    def _():
        pltpu.sync_copy(etab, etab_v)

    @pl.when(lax.axis_index("s") == 1)
    def _():
        pltpu.sync_copy(ntab, ntab_s)
    plsc.subcore_barrier()

    ebase = wid * EPW

    for blk in range(EPW // EB):
        for r in range(3):
            pltpu.sync_copy(ea_t.at[pl.ds(r * NE + ebase + blk * EB, EB)],
                            ea_c.at[pl.ds(r * EB, EB)])

        @pl.loop(0, EB // L)
        def _(i):
            a = ea_c[pl.ds(0 * EB + i * L, L)]
            b = ea_c[pl.ds(1 * EB + i * L, L)]
            c = ea_c[pl.ds(2 * EB + i * L, L)]
            flat = blk * EB + i * L
            eidx[(flat // EC), pl.ds((flat % EC) // L * L, L)] = a * 12 + b * 2 + c

    def _idx(j, n):
        return eidx.at[j] if n == EC else eidx.at[j, pl.ds(0, n)]

    def _gather(j, b, n=EC):
        pltpu.async_copy(etab_v.at[_idx(j, n)], erows[b].at[pl.ds(0, n)],
                         gsems[b])

    def _wait_g(j, b, n=EC):
        pltpu.make_async_copy(etab_v.at[_idx(j, n)],
                              erows[b].at[pl.ds(0, n)], gsems[b]).wait()

    def _write(j, b, n=EC):
        pltpu.async_copy(erows[b].at[pl.ds(0, n)],
                         edge_out.at[pl.ds(ebase + j * EC, n)], wsems[b])

    def _wait_w(j, b, n=EC):
        pltpu.make_async_copy(erows[b].at[pl.ds(0, n)],
                              edge_out.at[pl.ds(ebase + j * EC, n)],
                              wsems[b]).wait()

    _gather(0, 0)
    _gather(1, 1)
    _wait_g(0, 0)
    _write(0, 0)
    _gather(2, 2)
    _wait_g(1, 1)
    _write(1, 1)
    _wait_w(0, 0)
    _gather(3, 0)

    @pl.loop(0, 24)
    def _(k):
        for t in range(3):
            j = 2 + k * 3 + t
            b = (2 + t) % 3
            b2 = (2 + t + 2) % 3
            _wait_g(j, b)
            _write(j, b)
            _wait_w(j - 1, b2)
            _gather(j + 2, b2)

    for j in range(74, ECF):
        b = j % 3
        b2 = (j + 2) % 3
        _wait_g(j, b)
        _write(j, b)
        if j + 2 <= ECF:
            _wait_w(j - 1, b2)
            if j + 2 < ECF:
                _gather(j + 2, b2)
            else:
                _gather(ECF, b2, ECT)
    _wait_g(ECF, ECF % 3, ECT)
    _write(ECF, ECF % 3, ECT)
    for j in range(ECF - 2, ECF + 1):
        _wait_w(j, j % 3, EC if j < ECF else ECT)

    @pl.when(wid < NT)
    def _():
        nbase = wid * NPW
        for f in range(9):
            pltpu.sync_copy(x_t.at[pl.ds(f * NN + nbase, NPW)],
                            x_v.at[pl.ds(f * NPW, NPW)])

        @pl.loop(0, NCH * (NCC // L))
        def _(i):
            c = i // (NCC // L)
            v = i % (NCC // L)
            d = pl.ds(v * L, L)

            def xf(f):
                return x_v[pl.ds(f * NPW + c * NCC + v * L, L)]

            nidx[c * 4 + 0, d] = xf(0) * 4 + xf(7) * 2 + xf(8)
            nidx[c * 4 + 1, d] = xf(1) * 11 + xf(2) + OFF1
            nidx[c * 4 + 2, d] = xf(3) * 9 + xf(4) + OFF2
            nidx[c * 4 + 3, d] = xf(5) * 8 + xf(6) + OFF3

        for c in range(NCH):
            for g in range(4):
                pltpu.async_copy(ntab_s.at[nidx.at[c * 4 + g]], nbuf[g],
                                 nsem[g])
            for g in range(4):
                pltpu.make_async_copy(ntab_s.at[nidx.at[c * 4 + g]], nbuf[g],
                                      nsem[g]).wait()
                if g:
                    @pl.loop(0, NCC)
                    def _(r):
                        for u in range(H // L):
                            sl = pl.ds(u * L, L)
                            nb0[r, sl] = nb0[r, sl] + nbuf[g][r, sl]

            pltpu.sync_copy(nb0, node_out.at[pl.ds(nbase + c * NCC, NCC)])


def _sc_embed(x_t, ea_t, ntab, etab):
    mesh = plsc.VectorSubcoreMesh(core_axis_name="c", subcore_axis_name="s",
                                  num_cores=NC, num_subcores=NS)
    return pl.kernel(
        _sc_body,
        out_type=(jax.ShapeDtypeStruct((NN, H), jnp.float32),
                  jax.ShapeDtypeStruct((NE, H), jnp.float32)),
        mesh=mesh,
        compiler_params=pltpu.CompilerParams(needs_layout_passes=False),
        scratch_types=[
            pltpu.VMEM_SHARED((ETROWS, H), jnp.float32),
            pltpu.VMEM_SHARED((NTROWS, H), jnp.float32),
            pltpu.VMEM((ECF + 1, EC), jnp.int32),
            pltpu.VMEM((3 * EB,), jnp.int32),
            pltpu.VMEM((EC, H), jnp.float32),
            pltpu.VMEM((EC, H), jnp.float32),
            pltpu.VMEM((EC, H), jnp.float32),
            pltpu.VMEM((9 * NPW,), jnp.int32),
            pltpu.VMEM((4 * NCH, NCC), jnp.int32),
            pltpu.VMEM((NCC, H), jnp.float32),
            pltpu.VMEM((NCC, H), jnp.float32),
            pltpu.VMEM((NCC, H), jnp.float32),
            pltpu.VMEM((NCC, H), jnp.float32),
            pltpu.SemaphoreType.DMA,
            pltpu.SemaphoreType.DMA,
            pltpu.SemaphoreType.DMA,
            pltpu.SemaphoreType.DMA,
            pltpu.SemaphoreType.DMA,
            pltpu.SemaphoreType.DMA,
            pltpu.SemaphoreType.DMA,
            pltpu.SemaphoreType.DMA,
            pltpu.SemaphoreType.DMA,
            pltpu.SemaphoreType.DMA,
        ],
    )(x_t, ea_t, ntab, etab)


def kernel(x, edge_attr,
           node_emb_0, node_emb_1, node_emb_2, node_emb_3, node_emb_4,
           node_emb_5, node_emb_6, node_emb_7, node_emb_8,
           edge_emb_0, edge_emb_1, edge_emb_2):
    etab = (edge_emb_0[:, None, None, :] + edge_emb_1[None, :, None, :]
            + edge_emb_2[None, None, :, :]).reshape(-1, H)
    g0 = (node_emb_0[:, None, None, :] + node_emb_7[None, :, None, :]
          + node_emb_8[None, None, :, :]).reshape(-1, H)
    g1 = (node_emb_1[:, None, :] + node_emb_2[None, :, :]).reshape(-1, H)
    g2 = (node_emb_3[:, None, :] + node_emb_4[None, :, :]).reshape(-1, H)
    g3 = (node_emb_5[:, None, :] + node_emb_6[None, :, :]).reshape(-1, H)
    ntab = jnp.concatenate([g0, g1, g2, g3], axis=0)

    x_t = x.T.reshape(-1)
    ea_t = edge_attr.T.reshape(-1)
    node_out, edge_out = _sc_embed(x_t, ea_t, ntab, etab)
    return (node_out, edge_out)


# speedup vs baseline: 1.0038x; 1.0038x over previous
import jax
import jax.numpy as jnp
from jax import lax
from jax.experimental import pallas as pl
from jax.experimental.pallas import tpu as pltpu
from jax.experimental.pallas import tpu_sc as plsc

H = 128
NN = 10000
NE = 320000
NC, NS, L = 2, 16, 16
NW = NC * NS

EPW = NE // NW
EC = 128
ECF = EPW // EC
ECT = EPW - ECF * EC
EB = 2000
NB = 2

NT = 25
NPW = NN // NT
NCC = 80
NCH = NPW // NCC

ETROWS = 22 * 6 * 2
NTROWS = 476 + 99 + 108 + 40
OFF1, OFF2, OFF3 = 476, 575, 683


def _sc_body(x_t, ea_t, ntab, etab, node_out, edge_out,
             etab_v, ntab_s, eidx, ea_c, erows0, erows1, x_v, nidx,
             nb0, nb1, nb2, nb3,
             ws0, ws1, gs0, gs1, ns0, ns1, ns2, ns3):
    erows = (erows0, erows1)
    wsems = (ws0, ws1)
    nbuf = (nb0, nb1, nb2, nb3)
    nsem = (ns0, ns1, ns2, ns3)
    wid = lax.axis_index("s") * NC + lax.axis_index("c")

    @pl.when(lax.axis_index("s") == 0)
    def _():
        pltpu.sync_copy(etab, etab_v)

    @pl.when(lax.axis_index("s") == 1)
    def _():
        pltpu.sync_copy(ntab, ntab_s)
    plsc.subcore_barrier()

    ebase = wid * EPW

    for blk in range(EPW // EB):
        for r in range(3):
            pltpu.sync_copy(ea_t.at[pl.ds(r * NE + ebase + blk * EB, EB)],
                            ea_c.at[pl.ds(r * EB, EB)])

        @pl.loop(0, EB // L)
        def _(i):
            a = ea_c[pl.ds(0 * EB + i * L, L)]
            b = ea_c[pl.ds(1 * EB + i * L, L)]
            c = ea_c[pl.ds(2 * EB + i * L, L)]
            flat = blk * EB + i * L
            eidx[(flat // EC), pl.ds((flat % EC) // L * L, L)] = a * 12 + b * 2 + c

    def _idx(j, n):
        return eidx.at[j] if n == EC else eidx.at[j, pl.ds(0, n)]

    def _fill(j, b, n, sem):
        pltpu.async_copy(etab_v.at[_idx(j, n)], erows[b].at[pl.ds(0, n)],
                         sem)
        pltpu.make_async_copy(etab_v.at[_idx(j, n)],
                              erows[b].at[pl.ds(0, n)], sem).wait()

    def _write(j, b, n=EC):
        pltpu.async_copy(erows[b].at[pl.ds(0, n)],
                         edge_out.at[pl.ds(ebase + j * EC, n)], wsems[b])

    def _wait_w(j, b, n=EC):
        pltpu.make_async_copy(erows[b].at[pl.ds(0, n)],
                              edge_out.at[pl.ds(ebase + j * EC, n)],
                              wsems[b]).wait()

    _fill(0, 0, EC, gs0)
    _write(0, 0)
    _fill(1, 1, EC, gs1)
    _write(1, 1)

    @pl.loop(0, (ECF - 2) // NB)
    def _(k):
        for t in range(NB):
            j = 2 + k * NB + t
            _wait_w(j - 2, t)
            _fill(j, t, EC, (gs0, gs1)[t])
            _write(j, t)

    _wait_w(ECF - 2, 0)
    _fill(ECF, 0, ECT, gs0)
    _write(ECF, 0, ECT)
    _wait_w(ECF - 1, 1)
    _wait_w(ECF, 0, ECT)

    @pl.when(wid < NT)
    def _():
        nbase = wid * NPW
        for f in range(9):
            pltpu.sync_copy(x_t.at[pl.ds(f * NN + nbase, NPW)],
                            x_v.at[pl.ds(f * NPW, NPW)])

        @pl.loop(0, NCH * (NCC // L))
        def _(i):
            c = i // (NCC // L)
            v = i % (NCC // L)
            d = pl.ds(v * L, L)

            def xf(f):
                return x_v[pl.ds(f * NPW + c * NCC + v * L, L)]

            nidx[c * 4 + 0, d] = xf(0) * 4 + xf(7) * 2 + xf(8)
            nidx[c * 4 + 1, d] = xf(1) * 11 + xf(2) + OFF1
            nidx[c * 4 + 2, d] = xf(3) * 9 + xf(4) + OFF2
            nidx[c * 4 + 3, d] = xf(5) * 8 + xf(6) + OFF3

        for c in range(NCH):
            for g in range(4):
                pltpu.async_copy(ntab_s.at[nidx.at[c * 4 + g]], nbuf[g],
                                 nsem[g])
            for g in range(4):
                pltpu.make_async_copy(ntab_s.at[nidx.at[c * 4 + g]], nbuf[g],
                                      nsem[g]).wait()
                if g:
                    @pl.loop(0, NCC)
                    def _(r):
                        for u in range(H // L):
                            sl = pl.ds(u * L, L)
                            nb0[r, sl] = nb0[r, sl] + nbuf[g][r, sl]

            pltpu.sync_copy(nb0, node_out.at[pl.ds(nbase + c * NCC, NCC)])


def _sc_embed(x_t, ea_t, ntab, etab):
    mesh = plsc.VectorSubcoreMesh(core_axis_name="c", subcore_axis_name="s",
                                  num_cores=NC, num_subcores=NS)
    return pl.kernel(
        _sc_body,
        out_type=(jax.ShapeDtypeStruct((NN, H), jnp.float32),
                  jax.ShapeDtypeStruct((NE, H), jnp.float32)),
        mesh=mesh,
        compiler_params=pltpu.CompilerParams(needs_layout_passes=False),
        scratch_types=[
            pltpu.VMEM_SHARED((ETROWS, H), jnp.float32),
            pltpu.VMEM_SHARED((NTROWS, H), jnp.float32),
            pltpu.VMEM((ECF + 1, EC), jnp.int32),
            pltpu.VMEM((3 * EB,), jnp.int32),
            pltpu.VMEM((EC, H), jnp.float32),
            pltpu.VMEM((EC, H), jnp.float32),
            pltpu.VMEM((9 * NPW,), jnp.int32),
            pltpu.VMEM((4 * NCH, NCC), jnp.int32),
            pltpu.VMEM((NCC, H), jnp.float32),
            pltpu.VMEM((NCC, H), jnp.float32),
            pltpu.VMEM((NCC, H), jnp.float32),
            pltpu.VMEM((NCC, H), jnp.float32),
            pltpu.SemaphoreType.DMA,
            pltpu.SemaphoreType.DMA,
            pltpu.SemaphoreType.DMA,
            pltpu.SemaphoreType.DMA,
            pltpu.SemaphoreType.DMA,
            pltpu.SemaphoreType.DMA,
            pltpu.SemaphoreType.DMA,
            pltpu.SemaphoreType.DMA,
        ],
    )(x_t, ea_t, ntab, etab)


def kernel(x, edge_attr,
           node_emb_0, node_emb_1, node_emb_2, node_emb_3, node_emb_4,
           node_emb_5, node_emb_6, node_emb_7, node_emb_8,
           edge_emb_0, edge_emb_1, edge_emb_2):
    etab = (edge_emb_0[:, None, None, :] + edge_emb_1[None, :, None, :]
            + edge_emb_2[None, None, :, :]).reshape(-1, H)
    g0 = (node_emb_0[:, None, None, :] + node_emb_7[None, :, None, :]
          + node_emb_8[None, None, :, :]).reshape(-1, H)
    g1 = (node_emb_1[:, None, :] + node_emb_2[None, :, :]).reshape(-1, H)
    g2 = (node_emb_3[:, None, :] + node_emb_4[None, :, :]).reshape(-1, H)
    g3 = (node_emb_5[:, None, :] + node_emb_6[None, :, :]).reshape(-1, H)
    ntab = jnp.concatenate([g0, g1, g2, g3], axis=0)

    x_t = x.T.reshape(-1)
    ea_t = edge_attr.T.reshape(-1)
    node_out, edge_out = _sc_embed(x_t, ea_t, ntab, etab)
    return (node_out, edge_out)
